# Initial kernel scaffold; baseline (speedup 1.0000x reference)
import jax
import jax.numpy as jnp
from jax.experimental import pallas as pl


def kernel(output, target):
    raise NotImplementedError("write your pallas kernel here")


# speedup vs baseline: 5.6188x; 5.6188x over previous
import functools

import jax
import jax.numpy as jnp
from jax import lax
from jax.experimental import pallas as pl
from jax.experimental.pallas import tpu as pltpu
from jax.experimental.pallas import tpu_sc as plsc

N = 1048576
NUM_WORKERS = 32
ROWS_PER_WORKER = N // NUM_WORKERS
FLOATS_PER_WORKER = 2 * ROWS_PER_WORKER
LANES = 16
STEPS = ROWS_PER_WORKER // LANES


def _f1_counts_sc(xflat, target):
  mesh = plsc.VectorSubcoreMesh(core_axis_name="c", subcore_axis_name="s")

  @functools.partial(
      pl.kernel,
      mesh=mesh,
      out_type=jax.ShapeDtypeStruct((NUM_WORKERS, LANES), jnp.int32),
      scratch_types=[
          pltpu.VMEM((FLOATS_PER_WORKER,), jnp.float32),
          pltpu.VMEM((ROWS_PER_WORKER,), jnp.int32),
          pltpu.VMEM((LANES,), jnp.int32),
          pltpu.SemaphoreType.DMA,
          pltpu.SemaphoreType.DMA,
      ],
      compiler_params=pltpu.CompilerParams(needs_layout_passes=False),
  )
  def body(x_hbm, t_hbm, out_hbm, x_v, t_v, stage_v, sem_x, sem_t):
    wid = lax.axis_index("s") * 2 + lax.axis_index("c")

    cp_x = pltpu.async_copy(
        x_hbm.at[pl.ds(wid * FLOATS_PER_WORKER, FLOATS_PER_WORKER)],
        x_v, sem_x)
    cp_t = pltpu.async_copy(
        t_hbm.at[pl.ds(wid * ROWS_PER_WORKER, ROWS_PER_WORKER)],
        t_v, sem_t)
    cp_x.wait()
    cp_t.wait()

    lane = lax.iota(jnp.int32, LANES)
    even = lane * 2
    zero = jnp.zeros((LANES,), jnp.int32)
    one = jnp.ones((LANES,), jnp.int32)

    def step(i, carry):
      acc_tp, acc_p, acc_t = carry
      idx0 = even + i * (2 * LANES)
      c0 = plsc.load_gather(x_v, [idx0])
      c1 = plsc.load_gather(x_v, [idx0 + 1])
      tv = t_v[pl.ds(i * LANES, LANES)]
      pred = c1 > c0
      predi = jnp.where(pred, one, zero)
      tp_inc = jnp.where(pred, tv, zero)
      return (acc_tp + tp_inc, acc_p + predi, acc_t + tv)

    acc_tp, acc_p, acc_t = lax.fori_loop(
        0, STEPS, step, (zero, zero, zero), unroll=4)

    tp_s = jnp.sum(acc_tp)
    p_s = jnp.sum(acc_p)
    t_s = jnp.sum(acc_t)
    stage = jnp.where(lane == 0, tp_s,
                      jnp.where(lane == 1, p_s,
                                jnp.where(lane == 2, t_s, 0)))
    stage_v[...] = stage
    pltpu.sync_copy(stage_v, out_hbm.at[wid])

  return body(xflat, target)


@jax.jit
def kernel(output, target):
  xflat = output.reshape(-1)
  parts = _f1_counts_sc(xflat, target)
  sums = jnp.sum(parts, axis=0)
  tp = sums[0].astype(jnp.float32)
  fp = (sums[1] - sums[0]).astype(jnp.float32)
  fn = (sums[2] - sums[0]).astype(jnp.float32)
  precision = tp / (tp + fp + 1e-10)
  recall = tp / (tp + fn + 1e-10)
  return 2 * precision * recall / (precision + recall + 1e-10)


# speedup vs baseline: 238.4152x; 42.4318x over previous
import functools

import jax
import jax.numpy as jnp
from jax import lax
from jax.experimental import pallas as pl
from jax.experimental.pallas import tpu as pltpu
from jax.experimental.pallas import tpu_sc as plsc

N = 1048576
LANES = 16
BLK = 128
NUM_BLOCKS = N // BLK
NUM_WORKERS = 32
BLOCKS_PER_WORKER = NUM_BLOCKS // NUM_WORKERS
ROWS_PER_WORKER = N // NUM_WORKERS
VECS_PER_BLK = BLK // LANES


def _f1_counts_sc(xview, target):
  mesh = plsc.VectorSubcoreMesh(core_axis_name="c", subcore_axis_name="s")

  @functools.partial(
      pl.kernel,
      mesh=mesh,
      out_type=jax.ShapeDtypeStruct((NUM_WORKERS, LANES), jnp.int32),
      scratch_types=[
          pltpu.VMEM((BLOCKS_PER_WORKER, 2, BLK), jnp.float32),
          pltpu.VMEM((ROWS_PER_WORKER,), jnp.int32),
          pltpu.VMEM((LANES,), jnp.int32),
          pltpu.SemaphoreType.DMA,
          pltpu.SemaphoreType.DMA,
      ],
      compiler_params=pltpu.CompilerParams(needs_layout_passes=False),
  )
  def body(x_hbm, t_hbm, out_hbm, x_v, t_v, stage_v, sem_x, sem_t):
    wid = lax.axis_index("s") * 2 + lax.axis_index("c")

    cp_x = pltpu.async_copy(
        x_hbm.at[pl.ds(wid * BLOCKS_PER_WORKER, BLOCKS_PER_WORKER)],
        x_v, sem_x)
    cp_t = pltpu.async_copy(
        t_hbm.at[pl.ds(wid * ROWS_PER_WORKER, ROWS_PER_WORKER)],
        t_v, sem_t)
    cp_x.wait()
    cp_t.wait()

    lane = lax.iota(jnp.int32, LANES)
    zero = jnp.zeros((LANES,), jnp.int32)
    one = jnp.ones((LANES,), jnp.int32)

    def blk_step(k, carry):
      acc_tp, acc_p, acc_t = carry
      for j in range(VECS_PER_BLK):
        c0 = x_v[k, 0, pl.ds(j * LANES, LANES)]
        c1 = x_v[k, 1, pl.ds(j * LANES, LANES)]
        tv = t_v[pl.ds(k * BLK + j * LANES, LANES)]
        pred = c1 > c0
        acc_p = acc_p + jnp.where(pred, one, zero)
        acc_tp = acc_tp + jnp.where(pred, tv, zero)
        acc_t = acc_t + tv
      return (acc_tp, acc_p, acc_t)

    acc_tp, acc_p, acc_t = lax.fori_loop(
        0, BLOCKS_PER_WORKER, blk_step, (zero, zero, zero))

    tp_s = jnp.sum(acc_tp)
    p_s = jnp.sum(acc_p)
    t_s = jnp.sum(acc_t)
    stage = jnp.where(lane == 0, tp_s,
                      jnp.where(lane == 1, p_s,
                                jnp.where(lane == 2, t_s, 0)))
    stage_v[...] = stage
    pltpu.sync_copy(stage_v, out_hbm.at[wid])

  return body(xview, target)


@jax.jit
def kernel(output, target):
  xview = output.reshape(NUM_BLOCKS, BLK, 2).transpose(0, 2, 1)
  parts = _f1_counts_sc(xview, target)
  sums = jnp.sum(parts, axis=0)
  tp = sums[0].astype(jnp.float32)
  fp = (sums[1] - sums[0]).astype(jnp.float32)
  fn = (sums[2] - sums[0]).astype(jnp.float32)
  precision = tp / (tp + fp + 1e-10)
  recall = tp / (tp + fn + 1e-10)
  return 2 * precision * recall / (precision + recall + 1e-10)


# speedup vs baseline: 268.4717x; 1.1261x over previous
import functools

import jax
import jax.numpy as jnp
from jax import lax
from jax.experimental import pallas as pl
from jax.experimental.pallas import tpu as pltpu
from jax.experimental.pallas import tpu_sc as plsc

N = 1048576
LANES = 16
BLK = 128
NUM_BLOCKS = N // BLK
NUM_WORKERS = 32
BLOCKS_PER_WORKER = NUM_BLOCKS // NUM_WORKERS
ROWS_PER_WORKER = N // NUM_WORKERS
VECS_PER_BLK = BLK // LANES


def _f1_counts_sc(xview, target):
  mesh = plsc.VectorSubcoreMesh(core_axis_name="c", subcore_axis_name="s")

  @functools.partial(
      pl.kernel,
      mesh=mesh,
      out_type=jax.ShapeDtypeStruct((NUM_WORKERS, LANES), jnp.int32),
      scratch_types=[
          pltpu.VMEM((BLOCKS_PER_WORKER, 2, BLK), jnp.float32),
          pltpu.VMEM((ROWS_PER_WORKER,), jnp.int32),
          pltpu.VMEM((LANES,), jnp.int32),
          pltpu.SemaphoreType.DMA,
          pltpu.SemaphoreType.DMA,
      ],
      compiler_params=pltpu.CompilerParams(needs_layout_passes=False),
  )
  def body(x_hbm, t_hbm, out_hbm, x_v, t_v, stage_v, sem_x, sem_t):
    wid = lax.axis_index("s") * 2 + lax.axis_index("c")

    cp_x = pltpu.async_copy(
        x_hbm.at[pl.ds(wid * BLOCKS_PER_WORKER, BLOCKS_PER_WORKER)],
        x_v, sem_x)
    cp_t = pltpu.async_copy(
        t_hbm.at[pl.ds(wid * ROWS_PER_WORKER, ROWS_PER_WORKER)],
        t_v, sem_t)
    cp_x.wait()
    cp_t.wait()

    lane = lax.iota(jnp.int32, LANES)
    zero = jnp.zeros((LANES,), jnp.int32)
    one = jnp.ones((LANES,), jnp.int32)

    def blk_step(k, carry):
      acc_tp, acc_p, acc_t = carry
      for j in range(VECS_PER_BLK):
        c0 = x_v[k, 0, pl.ds(j * LANES, LANES)]
        c1 = x_v[k, 1, pl.ds(j * LANES, LANES)]
        tv = t_v[pl.ds(k * BLK + j * LANES, LANES)]
        pred = c1 > c0
        acc_p = acc_p + jnp.where(pred, one, zero)
        acc_tp = acc_tp + jnp.where(pred, tv, zero)
        acc_t = acc_t + tv
      return (acc_tp, acc_p, acc_t)

    acc_tp, acc_p, acc_t = lax.fori_loop(
        0, BLOCKS_PER_WORKER, blk_step, (zero, zero, zero))

    tp_s = jnp.sum(acc_tp)
    p_s = jnp.sum(acc_p)
    t_s = jnp.sum(acc_t)
    stage = jnp.where(lane == 0, tp_s,
                      jnp.where(lane == 1, p_s,
                                jnp.where(lane == 2, t_s, 0)))
    stage_v[...] = stage
    pltpu.sync_copy(stage_v, out_hbm.at[wid])

  return body(xview, target)


def _finalize_tc(parts):
  def fin(parts_ref, out_ref):
    x = parts_ref[...]
    col = lax.broadcasted_iota(jnp.int32, (NUM_WORKERS, LANES), 1)
    zero = jnp.zeros((NUM_WORKERS, LANES), jnp.int32)
    tp = jnp.sum(jnp.where(col == 0, x, zero)).astype(jnp.float32)
    p = jnp.sum(jnp.where(col == 1, x, zero)).astype(jnp.float32)
    t = jnp.sum(jnp.where(col == 2, x, zero)).astype(jnp.float32)
    fp = p - tp
    fn = t - tp
    precision = tp / (tp + fp + 1e-10)
    recall = tp / (tp + fn + 1e-10)
    out_ref[0, 0] = 2 * precision * recall / (precision + recall + 1e-10)

  out = pl.pallas_call(
      fin,
      out_shape=jax.ShapeDtypeStruct((1, 1), jnp.float32),
      out_specs=pl.BlockSpec(memory_space=pltpu.SMEM),
  )(parts)
  return out.reshape(())


@jax.jit
def kernel(output, target):
  xview = output.reshape(NUM_BLOCKS, BLK, 2).transpose(0, 2, 1)
  parts = _f1_counts_sc(xview, target)
  return _finalize_tc(parts)


# speedup vs baseline: 275.3404x; 1.0256x over previous
import functools

import jax
import jax.numpy as jnp
from jax import lax
from jax.experimental import pallas as pl
from jax.experimental.pallas import tpu as pltpu
from jax.experimental.pallas import tpu_sc as plsc

N = 1048576
LANES = 16
BLK = 128
NUM_BLOCKS = N // BLK
NUM_WORKERS = 32
BLOCKS_PER_WORKER = NUM_BLOCKS // NUM_WORKERS
ROWS_PER_WORKER = N // NUM_WORKERS
VECS_PER_BLK = BLK // LANES
NUM_CHUNKS = 8
BLOCKS_PER_CHUNK = BLOCKS_PER_WORKER // NUM_CHUNKS
ROWS_PER_CHUNK = ROWS_PER_WORKER // NUM_CHUNKS


def _f1_counts_sc(xview, target):
  mesh = plsc.VectorSubcoreMesh(core_axis_name="c", subcore_axis_name="s")

  @functools.partial(
      pl.kernel,
      mesh=mesh,
      out_type=jax.ShapeDtypeStruct((NUM_WORKERS, LANES), jnp.int32),
      scratch_types=[
          pltpu.VMEM((BLOCKS_PER_WORKER, 2, BLK), jnp.float32),
          pltpu.VMEM((ROWS_PER_WORKER,), jnp.int32),
          pltpu.VMEM((LANES,), jnp.int32),
      ]
      + [pltpu.SemaphoreType.DMA] * (2 * NUM_CHUNKS),
      compiler_params=pltpu.CompilerParams(needs_layout_passes=False),
  )
  def body(x_hbm, t_hbm, out_hbm, x_v, t_v, stage_v, *sems):
    wid = lax.axis_index("s") * 2 + lax.axis_index("c")
    blk0 = wid * BLOCKS_PER_WORKER
    row0 = wid * ROWS_PER_WORKER

    copies = []
    for g in range(NUM_CHUNKS):
      cp_x = pltpu.async_copy(
          x_hbm.at[pl.ds(blk0 + g * BLOCKS_PER_CHUNK, BLOCKS_PER_CHUNK)],
          x_v.at[pl.ds(g * BLOCKS_PER_CHUNK, BLOCKS_PER_CHUNK)],
          sems[2 * g])
      cp_t = pltpu.async_copy(
          t_hbm.at[pl.ds(row0 + g * ROWS_PER_CHUNK, ROWS_PER_CHUNK)],
          t_v.at[pl.ds(g * ROWS_PER_CHUNK, ROWS_PER_CHUNK)],
          sems[2 * g + 1])
      copies.append((cp_x, cp_t))

    lane = lax.iota(jnp.int32, LANES)
    zero = jnp.zeros((LANES,), jnp.int32)
    one = jnp.ones((LANES,), jnp.int32)

    def blk_step(k, carry):
      acc_tp, acc_p, acc_t = carry
      for j in range(VECS_PER_BLK):
        c0 = x_v[k, 0, pl.ds(j * LANES, LANES)]
        c1 = x_v[k, 1, pl.ds(j * LANES, LANES)]
        tv = t_v[pl.ds(k * BLK + j * LANES, LANES)]
        pred = c1 > c0
        acc_p = acc_p + jnp.where(pred, one, zero)
        acc_tp = acc_tp + jnp.where(pred, tv, zero)
        acc_t = acc_t + tv
      return (acc_tp, acc_p, acc_t)

    acc = (zero, zero, zero)
    for g in range(NUM_CHUNKS):
      cp_x, cp_t = copies[g]
      cp_x.wait()
      cp_t.wait()
      acc = lax.fori_loop(
          g * BLOCKS_PER_CHUNK, (g + 1) * BLOCKS_PER_CHUNK, blk_step, acc)
    acc_tp, acc_p, acc_t = acc

    tp_s = jnp.sum(acc_tp)
    p_s = jnp.sum(acc_p)
    t_s = jnp.sum(acc_t)
    stage = jnp.where(lane == 0, tp_s,
                      jnp.where(lane == 1, p_s,
                                jnp.where(lane == 2, t_s, 0)))
    stage_v[...] = stage
    pltpu.sync_copy(stage_v, out_hbm.at[wid])

  return body(xview, target)


def _finalize_tc(parts):
  def fin(parts_ref, out_ref):
    x = parts_ref[...]
    col = lax.broadcasted_iota(jnp.int32, (NUM_WORKERS, LANES), 1)
    zero = jnp.zeros((NUM_WORKERS, LANES), jnp.int32)
    tp = jnp.sum(jnp.where(col == 0, x, zero)).astype(jnp.float32)
    p = jnp.sum(jnp.where(col == 1, x, zero)).astype(jnp.float32)
    t = jnp.sum(jnp.where(col == 2, x, zero)).astype(jnp.float32)
    fp = p - tp
    fn = t - tp
    precision = tp / (tp + fp + 1e-10)
    recall = tp / (tp + fn + 1e-10)
    out_ref[0, 0] = 2 * precision * recall / (precision + recall + 1e-10)

  out = pl.pallas_call(
      fin,
      out_shape=jax.ShapeDtypeStruct((1, 1), jnp.float32),
      out_specs=pl.BlockSpec(memory_space=pltpu.SMEM),
  )(parts)
  return out.reshape(())


@jax.jit
def kernel(output, target):
  xview = output.reshape(NUM_BLOCKS, BLK, 2).transpose(0, 2, 1)
  parts = _f1_counts_sc(xview, target)
  return _finalize_tc(parts)
